# baseline (speedup 1.0000x reference)
import functools
import math

import jax
import jax.numpy as jnp
from jax import lax
from jax.experimental import pallas as pl
from jax.experimental.pallas import tpu as pltpu
from jax.experimental.pallas import tpu_sc as plsc

EMBED = 24
_C = 4096


def _pad_to(n: int, mult: int) -> int:
    return ((n + mult - 1) // mult) * mult


@functools.cache
def _msg_sc(NNp: int, nchunks: int):
    C = _C
    cpt = nchunks // 16
    Ep = nchunks * C
    mesh = plsc.VectorSubcoreMesh(core_axis_name="c", subcore_axis_name="s")

    @functools.partial(
        pl.kernel,
        out_type=jax.ShapeDtypeStruct((2, Ep, 16), jnp.float32),
        mesh=mesh,
        compiler_params=pltpu.CompilerParams(
            needs_layout_passes=False, use_tc_tiling_on_sc=False),
        scratch_types=[
            pltpu.VMEM((C,), jnp.int32),
            pltpu.VMEM((C,), jnp.float32),
            pltpu.VMEM((C, 16), jnp.float32),
            pltpu.SemaphoreType.DMA,
        ],
    )
    def k(m2, srch, ewh, out, src_v, ew_v, rows_v, sem):
        c = lax.axis_index("c")
        s = lax.axis_index("s")

        def chunk_body(t, carry):
            base = (t * 16 + s) * C
            pltpu.sync_copy(srch.at[pl.ds(base, C)], src_v)
            pltpu.sync_copy(ewh.at[pl.ds(base, C)], ew_v)

            @pl.when(c == 0)
            def _():
                pltpu.async_copy(m2.at[0].at[src_v], rows_v, sem).wait()

            @pl.when(c == 1)
            def _():
                pltpu.async_copy(m2.at[1].at[src_v], rows_v, sem).wait()

            def mul_body(gi, carry2):
                eb = gi * 16
                for j in range(16):
                    w = plsc.load_gather(
                        ew_v, [jnp.full((16,), eb + j, jnp.int32)])
                    rows_v[eb + j] = rows_v[eb + j] * w
                return carry2
            lax.fori_loop(0, C // 16, mul_body, 0)

            @pl.when(c == 0)
            def _():
                pltpu.sync_copy(rows_v, out.at[0].at[pl.ds(base, C)])

            @pl.when(c == 1)
            def _():
                pltpu.sync_copy(rows_v, out.at[1].at[pl.ds(base, C)])
            return carry
        lax.fori_loop(0, cpt, chunk_body, 0)

    return k


def _msg(m, src, ew):
    NN = m.shape[0]
    E = src.shape[0]
    NNp = _pad_to(NN, 16)
    Ep = _pad_to(E, _C * 16)
    m2 = jnp.zeros((2, NNp, 16), jnp.float32)
    m2 = m2.at[0, :NN, :12].set(m[:, :12])
    m2 = m2.at[1, :NN, :12].set(m[:, 12:])
    srcp = jnp.pad(src, (0, Ep - E))
    ewp = jnp.pad(ew, (0, Ep - E))
    o = _msg_sc(NNp, Ep // _C)(m2, srcp, ewp)
    return jnp.concatenate([o[0, :E, :12], o[1, :E, :12]], axis=1)


_C2 = 1024


@functools.cache
def _remap_sc(Nm: int, nchunks: int):
    C = _C2
    cpt = nchunks // 32
    Ep = nchunks * C
    NmP = _pad_to(Nm, 16)
    mesh = plsc.VectorSubcoreMesh(core_axis_name="c", subcore_axis_name="s")

    @functools.partial(
        pl.kernel,
        out_type=(jax.ShapeDtypeStruct((Ep,), jnp.int32),
                  jax.ShapeDtypeStruct((Ep,), jnp.int32),
                  jax.ShapeDtypeStruct((Ep,), jnp.float32)),
        mesh=mesh,
        compiler_params=pltpu.CompilerParams(
            needs_layout_passes=False, use_tc_tiling_on_sc=False),
        scratch_types=[
            pltpu.VMEM((NmP,), jnp.int32),
            pltpu.VMEM((C,), jnp.int32),
            pltpu.VMEM((C,), jnp.int32),
            pltpu.VMEM((C,), jnp.float32),
            pltpu.VMEM((C,), jnp.int32),
            pltpu.VMEM((C,), jnp.int32),
            pltpu.VMEM((C,), jnp.float32),
        ],
    )
    def k(maph, srch, dsth, eah, ns_o, nd_o, ea_o,
          map_v, src_v, dst_v, ea_v, ns_v, nd_v, eao_v):
        c = lax.axis_index("c")
        s = lax.axis_index("s")
        pltpu.sync_copy(maph, map_v.at[pl.ds(0, Nm)])
        w = s * 2 + c
        zero = jnp.zeros((16,), jnp.float32)

        def chunk_body(t, carry):
            base = (t * 32 + w) * C
            pltpu.sync_copy(srch.at[pl.ds(base, C)], src_v)
            pltpu.sync_copy(dsth.at[pl.ds(base, C)], dst_v)
            pltpu.sync_copy(eah.at[pl.ds(base, C)], ea_v)

            def grp(gi, carry2):
                eb = gi * 16
                sl = pl.ds(eb, 16)
                ns = plsc.load_gather(map_v, [src_v[sl]])
                nd = plsc.load_gather(map_v, [dst_v[sl]])
                valid = (ns >= 0) & (nd >= 0)
                zero_i = jnp.zeros((16,), jnp.int32)
                ns_v[sl] = jnp.where(valid, ns, zero_i)
                nd_v[sl] = jnp.where(valid, nd, zero_i)
                eao_v[sl] = jnp.where(valid, ea_v[sl], zero)
                return carry2
            lax.fori_loop(0, C // 16, grp, 0)
            pltpu.sync_copy(ns_v, ns_o.at[pl.ds(base, C)])
            pltpu.sync_copy(nd_v, nd_o.at[pl.ds(base, C)])
            pltpu.sync_copy(eao_v, ea_o.at[pl.ds(base, C)])
            return carry
        lax.fori_loop(0, cpt, chunk_body, 0)

    return k


def _remap(mapping, src, dst, ea):
    E = src.shape[0]
    Ep = _pad_to(E, _C2 * 32)
    srcp = jnp.pad(src, (0, Ep - E))
    dstp = jnp.pad(dst, (0, Ep - E))
    eap = jnp.pad(ea, (0, Ep - E))
    ns, nd, ean = _remap_sc(mapping.shape[0], Ep // _C2)(mapping, srcp, dstp, eap)
    return ns[:E], nd[:E], ean[:E]


def _gru(m, h, Wih, Whh, bih, bhh):
    gi = m @ Wih.T + bih
    gh = h @ Whh.T + bhh
    ir, iz, inn = jnp.split(gi, 3, axis=1)
    hr, hz, hn = jnp.split(gh, 3, axis=1)
    r = jax.nn.sigmoid(ir + hr)
    z = jax.nn.sigmoid(iz + hz)
    n = jnp.tanh(inn + r * hn)
    return (1.0 - z) * n + z * h


def _ggc(x, src, dst, ew, p, pre):
    W = p[pre + '_W']
    for i in range(2):
        m = x @ W[i]
        msg = _msg(m, src, ew)
        agg = jnp.zeros_like(x).at[dst].add(msg)
        x = _gru(agg, x, p[pre + '_Wih'], p[pre + '_Whh'], p[pre + '_bih'], p[pre + '_bhh'])
    return x


def _topk_pool(x, src, dst, ew, w, Bn, n_per, ratio):
    k = math.ceil(n_per * ratio)
    score = jnp.tanh((x @ w) / jnp.linalg.norm(w))
    sv, si = jax.lax.top_k(score.reshape(Bn, n_per), k)
    perm = (si + (jnp.arange(Bn) * n_per)[:, None]).reshape(-1)
    xn = x[perm] * score[perm][:, None]
    mapping = jnp.full((Bn * n_per,), -1, dtype=jnp.int32).at[perm].set(
        jnp.arange(Bn * k, dtype=jnp.int32))
    ns, nd, ewn = _remap(mapping, src, dst, ew)
    return xn, ns, nd, ewn


def _set2set(x, Bn, n_per, p):
    xr = x.reshape(Bn, n_per, EMBED)
    h = jnp.zeros((Bn, EMBED), dtype=x.dtype)
    c = jnp.zeros((Bn, EMBED), dtype=x.dtype)
    q_star = jnp.zeros((Bn, 2 * EMBED), dtype=x.dtype)
    for _ in range(2):
        g = q_star @ p['lstm_Wih'].T + p['lstm_bih'] + h @ p['lstm_Whh'].T + p['lstm_bhh']
        ii, ff, gg, oo = jnp.split(g, 4, axis=1)
        ii = jax.nn.sigmoid(ii)
        ff = jax.nn.sigmoid(ff)
        gg = jnp.tanh(gg)
        oo = jax.nn.sigmoid(oo)
        c = ff * c + ii * gg
        h = oo * jnp.tanh(c)
        q = h
        e = (xr * q[:, None, :]).sum(-1)
        a = jax.nn.softmax(e, axis=1)
        r = (a[..., None] * xr).sum(1)
        q_star = jnp.concatenate([q, r], axis=1)
    return q_star


def kernel(x, edge_attr, y, edge_index, batch, params):
    Bn = y.shape[0]
    n0 = x.shape[0] // Bn
    indices = jnp.tile(jnp.arange(n0), Bn)
    src, dst, ew = edge_index[0], edge_index[1], edge_attr[:, 0]
    x = jax.nn.relu(_ggc(x, src, dst, ew, params, 'conv1'))
    x, src, dst, ew = _topk_pool(x, src, dst, ew, params['pool1_w'], Bn, n0, 0.8)
    n1 = x.shape[0] // Bn
    x = jax.nn.relu(_ggc(x, src, dst, ew, params, 'conv2'))
    x, src, dst, ew = _topk_pool(x, src, dst, ew, params['pool2_w'], Bn, n1, 0.8)
    n2 = x.shape[0] // Bn
    x = jax.nn.relu(_ggc(x, src, dst, ew, params, 'conv3'))
    x, src, dst, ew = _topk_pool(x, src, dst, ew, params['pool3_w'], Bn, n2, 0.3)
    n3 = x.shape[0] // Bn
    x = jax.nn.relu(_ggc(x, src, dst, ew, params, 'conv4'))
    x = jax.nn.relu(_ggc(x, src, dst, ew, params, 'conv5'))
    x = jax.nn.relu(_ggc(x, src, dst, ew, params, 'conv6'))
    xr = x.reshape(Bn, n3, EMBED)
    gmp = xr.max(axis=1)
    gap = xr.mean(axis=1)
    s2s = _set2set(x, Bn, n3, params)
    x6 = jnp.concatenate([gmp, gap, s2s], axis=1)
    out = jax.nn.relu(x6 @ params['lin1_W'].T + params['lin1_b'])
    return out, indices


# speedup vs baseline: 1.5210x; 1.5210x over previous
import functools
import math

import jax
import jax.numpy as jnp
from jax import lax
from jax.experimental import pallas as pl
from jax.experimental.pallas import tpu as pltpu
from jax.experimental.pallas import tpu_sc as plsc

EMBED = 24
_C = 4096


def _pad_to(n: int, mult: int) -> int:
    return ((n + mult - 1) // mult) * mult


@functools.cache
def _msg_sc(NNp: int, nchunks: int):
    C = _C
    cpt = nchunks // 16
    Ep = nchunks * C
    mesh = plsc.VectorSubcoreMesh(core_axis_name="c", subcore_axis_name="s")

    @functools.partial(
        pl.kernel,
        out_type=jax.ShapeDtypeStruct((2, Ep, 16), jnp.float32),
        mesh=mesh,
        compiler_params=pltpu.CompilerParams(
            needs_layout_passes=False, use_tc_tiling_on_sc=False),
        scratch_types=[
            pltpu.VMEM((C,), jnp.int32),
            pltpu.VMEM((C,), jnp.float32),
            pltpu.VMEM((C, 16), jnp.float32),
            pltpu.VMEM((1, 16), jnp.float32),
            pltpu.SemaphoreType.DMA,
        ],
    )
    def k(m2, srch, ewh, out, src_v, ew_v, rows_v, zrow_v, sem):
        c = lax.axis_index("c")
        s = lax.axis_index("s")
        I16 = lax.iota(jnp.int32, 16)

        @pl.when(c == 0)
        def _():
            pltpu.sync_copy(m2.at[0].at[pl.ds(0, 1)], zrow_v)

        @pl.when(c == 1)
        def _():
            pltpu.sync_copy(m2.at[1].at[pl.ds(0, 1)], zrow_v)

        def chunk_body(t, carry):
            base = (t * 16 + s) * C
            pltpu.sync_copy(srch.at[pl.ds(base, C)], src_v)
            pltpu.sync_copy(ewh.at[pl.ds(base, C)], ew_v)

            def spread_body(gi, carry2):
                sl = pl.ds(gi * 16, 16)
                dummy = (gi * 16 + I16) & 8191
                src_v[sl] = jnp.where(ew_v[sl] == 0.0, dummy, src_v[sl])
                return carry2
            lax.fori_loop(0, C // 16, spread_body, 0)

            @pl.when(c == 0)
            def _():
                pltpu.async_copy(m2.at[0].at[src_v], rows_v, sem).wait()

            @pl.when(c == 1)
            def _():
                pltpu.async_copy(m2.at[1].at[src_v], rows_v, sem).wait()

            zrow = zrow_v[0] * 0.0

            def mul_body(gi, carry2):
                eb = gi * 16
                for j in range(16):
                    w = plsc.load_gather(
                        ew_v, [jnp.full((16,), eb + j, jnp.int32)])
                    rows_v[eb + j] = jnp.where(
                        w == 0.0, zrow, rows_v[eb + j] * w)
                return carry2
            lax.fori_loop(0, C // 16, mul_body, 0)

            @pl.when(c == 0)
            def _():
                pltpu.sync_copy(rows_v, out.at[0].at[pl.ds(base, C)])

            @pl.when(c == 1)
            def _():
                pltpu.sync_copy(rows_v, out.at[1].at[pl.ds(base, C)])
            return carry
        lax.fori_loop(0, cpt, chunk_body, 0)

    return k


def _msg(m, src, ew):
    NN = m.shape[0]
    E = src.shape[0]
    NNp = _pad_to(NN, 16)
    Ep = _pad_to(E, _C * 16)
    m2 = jnp.zeros((2, NNp, 16), jnp.float32)
    m2 = m2.at[0, :NN, :12].set(m[:, :12])
    m2 = m2.at[1, :NN, :12].set(m[:, 12:])
    srcp = jnp.pad(src, (0, Ep - E))
    ewp = jnp.pad(ew, (0, Ep - E))
    o = _msg_sc(NNp, Ep // _C)(m2, srcp, ewp)
    return jnp.concatenate([o[0, :E, :12], o[1, :E, :12]], axis=1)


_C2 = 1024


@functools.cache
def _remap_sc(Nm: int, nchunks: int):
    C = _C2
    cpt = nchunks // 32
    Ep = nchunks * C
    NmP = _pad_to(Nm, 16)
    mesh = plsc.VectorSubcoreMesh(core_axis_name="c", subcore_axis_name="s")

    @functools.partial(
        pl.kernel,
        out_type=(jax.ShapeDtypeStruct((Ep,), jnp.int32),
                  jax.ShapeDtypeStruct((Ep,), jnp.int32),
                  jax.ShapeDtypeStruct((Ep,), jnp.float32)),
        mesh=mesh,
        compiler_params=pltpu.CompilerParams(
            needs_layout_passes=False, use_tc_tiling_on_sc=False),
        scratch_types=[
            pltpu.VMEM((NmP,), jnp.int32),
            pltpu.VMEM((C,), jnp.int32),
            pltpu.VMEM((C,), jnp.int32),
            pltpu.VMEM((C,), jnp.float32),
            pltpu.VMEM((C,), jnp.int32),
            pltpu.VMEM((C,), jnp.int32),
            pltpu.VMEM((C,), jnp.float32),
        ],
    )
    def k(maph, srch, dsth, eah, ns_o, nd_o, ea_o,
          map_v, src_v, dst_v, ea_v, ns_v, nd_v, eao_v):
        c = lax.axis_index("c")
        s = lax.axis_index("s")
        pltpu.sync_copy(maph, map_v.at[pl.ds(0, Nm)])
        w = s * 2 + c
        zero = jnp.zeros((16,), jnp.float32)

        def chunk_body(t, carry):
            base = (t * 32 + w) * C
            pltpu.sync_copy(srch.at[pl.ds(base, C)], src_v)
            pltpu.sync_copy(dsth.at[pl.ds(base, C)], dst_v)
            pltpu.sync_copy(eah.at[pl.ds(base, C)], ea_v)

            def grp(gi, carry2):
                eb = gi * 16
                sl = pl.ds(eb, 16)
                ns = plsc.load_gather(map_v, [src_v[sl]])
                nd = plsc.load_gather(map_v, [dst_v[sl]])
                valid = (ns >= 0) & (nd >= 0)
                zero_i = jnp.zeros((16,), jnp.int32)
                ns_v[sl] = jnp.where(valid, ns, zero_i)
                nd_v[sl] = jnp.where(valid, nd, zero_i)
                eao_v[sl] = jnp.where(valid, ea_v[sl], zero)
                return carry2
            lax.fori_loop(0, C // 16, grp, 0)
            pltpu.sync_copy(ns_v, ns_o.at[pl.ds(base, C)])
            pltpu.sync_copy(nd_v, nd_o.at[pl.ds(base, C)])
            pltpu.sync_copy(eao_v, ea_o.at[pl.ds(base, C)])
            return carry
        lax.fori_loop(0, cpt, chunk_body, 0)

    return k


def _remap(mapping, src, dst, ea):
    E = src.shape[0]
    Ep = _pad_to(E, _C2 * 32)
    srcp = jnp.pad(src, (0, Ep - E))
    dstp = jnp.pad(dst, (0, Ep - E))
    eap = jnp.pad(ea, (0, Ep - E))
    ns, nd, ean = _remap_sc(mapping.shape[0], Ep // _C2)(mapping, srcp, dstp, eap)
    return ns[:E], nd[:E], ean[:E]


def _gru(m, h, Wih, Whh, bih, bhh):
    gi = m @ Wih.T + bih
    gh = h @ Whh.T + bhh
    ir, iz, inn = jnp.split(gi, 3, axis=1)
    hr, hz, hn = jnp.split(gh, 3, axis=1)
    r = jax.nn.sigmoid(ir + hr)
    z = jax.nn.sigmoid(iz + hz)
    n = jnp.tanh(inn + r * hn)
    return (1.0 - z) * n + z * h


def _ggc(x, src, dst, ew, p, pre):
    W = p[pre + '_W']
    for i in range(2):
        m = x @ W[i]
        msg = _msg(m, src, ew)
        agg = jnp.zeros_like(x).at[dst].add(msg)
        x = _gru(agg, x, p[pre + '_Wih'], p[pre + '_Whh'], p[pre + '_bih'], p[pre + '_bhh'])
    return x


def _topk_pool(x, src, dst, ew, w, Bn, n_per, ratio):
    k = math.ceil(n_per * ratio)
    score = jnp.tanh((x @ w) / jnp.linalg.norm(w))
    sv, si = jax.lax.top_k(score.reshape(Bn, n_per), k)
    perm = (si + (jnp.arange(Bn) * n_per)[:, None]).reshape(-1)
    xn = x[perm] * score[perm][:, None]
    mapping = jnp.full((Bn * n_per,), -1, dtype=jnp.int32).at[perm].set(
        jnp.arange(Bn * k, dtype=jnp.int32))
    ns, nd, ewn = _remap(mapping, src, dst, ew)
    return xn, ns, nd, ewn


def _set2set(x, Bn, n_per, p):
    xr = x.reshape(Bn, n_per, EMBED)
    h = jnp.zeros((Bn, EMBED), dtype=x.dtype)
    c = jnp.zeros((Bn, EMBED), dtype=x.dtype)
    q_star = jnp.zeros((Bn, 2 * EMBED), dtype=x.dtype)
    for _ in range(2):
        g = q_star @ p['lstm_Wih'].T + p['lstm_bih'] + h @ p['lstm_Whh'].T + p['lstm_bhh']
        ii, ff, gg, oo = jnp.split(g, 4, axis=1)
        ii = jax.nn.sigmoid(ii)
        ff = jax.nn.sigmoid(ff)
        gg = jnp.tanh(gg)
        oo = jax.nn.sigmoid(oo)
        c = ff * c + ii * gg
        h = oo * jnp.tanh(c)
        q = h
        e = (xr * q[:, None, :]).sum(-1)
        a = jax.nn.softmax(e, axis=1)
        r = (a[..., None] * xr).sum(1)
        q_star = jnp.concatenate([q, r], axis=1)
    return q_star


def kernel(x, edge_attr, y, edge_index, batch, params):
    Bn = y.shape[0]
    n0 = x.shape[0] // Bn
    indices = jnp.tile(jnp.arange(n0), Bn)
    src, dst, ew = edge_index[0], edge_index[1], edge_attr[:, 0]
    x = jax.nn.relu(_ggc(x, src, dst, ew, params, 'conv1'))
    x, src, dst, ew = _topk_pool(x, src, dst, ew, params['pool1_w'], Bn, n0, 0.8)
    n1 = x.shape[0] // Bn
    x = jax.nn.relu(_ggc(x, src, dst, ew, params, 'conv2'))
    x, src, dst, ew = _topk_pool(x, src, dst, ew, params['pool2_w'], Bn, n1, 0.8)
    n2 = x.shape[0] // Bn
    x = jax.nn.relu(_ggc(x, src, dst, ew, params, 'conv3'))
    x, src, dst, ew = _topk_pool(x, src, dst, ew, params['pool3_w'], Bn, n2, 0.3)
    n3 = x.shape[0] // Bn
    x = jax.nn.relu(_ggc(x, src, dst, ew, params, 'conv4'))
    x = jax.nn.relu(_ggc(x, src, dst, ew, params, 'conv5'))
    x = jax.nn.relu(_ggc(x, src, dst, ew, params, 'conv6'))
    xr = x.reshape(Bn, n3, EMBED)
    gmp = xr.max(axis=1)
    gap = xr.mean(axis=1)
    s2s = _set2set(x, Bn, n3, params)
    x6 = jnp.concatenate([gmp, gap, s2s], axis=1)
    out = jax.nn.relu(x6 @ params['lin1_W'].T + params['lin1_b'])
    return out, indices
